# baseline (speedup 1.0000x reference)
import functools

import jax
import jax.numpy as jnp
from jax import lax
from jax.experimental import pallas as pl
from jax.experimental.pallas import tpu as pltpu
from jax.experimental.pallas import tpu_sc as plsc

_C1 = 2246822507
_C2 = 3266489909


def _u32(v):
    return jnp.uint32(v)


def _mulhi_u32(a, b_const):
    a0 = a & _u32(0xFFFF)
    a1 = a >> _u32(16)
    b0 = _u32(b_const & 0xFFFF)
    b1 = _u32(b_const >> 16)
    t = a0 * b0
    t1 = a1 * b0 + (t >> _u32(16))
    t2 = a0 * b1 + (t1 & _u32(0xFFFF))
    return a1 * b1 + (t1 >> _u32(16)) + (t2 >> _u32(16))


def _floor_i32(x):
    t = x.astype(jnp.int32)
    tf = t.astype(jnp.float32)
    return t - jnp.where(tf > x, jnp.int32(1), jnp.int32(0))


def _bitmix_idx(x, y, mask):
    ix = _floor_i32(x)
    iy = _floor_i32(y)
    ixu = ix.astype(jnp.uint32)
    iyu = iy.astype(jnp.uint32)
    s = (ix >> 31).astype(jnp.uint32)
    d = ixu ^ ((ixu >> _u32(16)) | (s << _u32(16)))
    e_lo = d * _u32(_C1)
    e_hi = _mulhi_u32(d, _C1)
    f_lo = e_lo ^ ((e_lo >> _u32(13)) | (e_hi << _u32(19)))
    f_hi = e_hi ^ (e_hi >> _u32(13))
    t_lo = iyu * _u32(_C2)
    t_hi = _mulhi_u32(iyu, _C2) + jnp.where(
        iy < 0, _u32((1 << 32) - _C2), _u32(0))
    g_lo = f_lo + t_lo
    carry = jnp.where(g_lo < t_lo, _u32(1), _u32(0))
    g_hi = f_hi + t_hi + carry
    out = (g_lo ^ ((g_lo >> _u32(16)) | (g_hi << _u32(16)))) & _u32(mask)
    return out.astype(jnp.int32)


@functools.cache
def _build(n, hash_size, dim):
    assert hash_size & (hash_size - 1) == 0
    info = plsc.get_sparse_core_info()
    nc, ns, lanes = info.num_cores, info.num_subcores, info.num_lanes
    nw = nc * ns
    b_per_w = n // nw
    assert b_per_w * nw == n
    chunk = min(128, b_per_w)
    nchunk = b_per_w // chunk
    assert nchunk * chunk == b_per_w

    mesh = plsc.VectorSubcoreMesh(core_axis_name="c", subcore_axis_name="s")

    @functools.partial(
        pl.kernel,
        out_type=jax.ShapeDtypeStruct((nw, nchunk, chunk, dim), jnp.float32),
        mesh=mesh,
        compiler_params=pltpu.CompilerParams(use_tc_tiling_on_sc=False),
        scratch_types=[
            pltpu.VMEM((b_per_w,), jnp.float32),
            pltpu.VMEM((b_per_w,), jnp.float32),
            pltpu.VMEM((nchunk, chunk), jnp.int32),
            pltpu.VMEM((nchunk, chunk, dim), jnp.float32),
            pltpu.SemaphoreType.DMA,
        ],
    )
    def grid_hash_gather(px_hbm, py_hbm, table_hbm, out_hbm,
                         px_v, py_v, idx_v, rows_v, sem):
        wid = lax.axis_index("s") * nc + lax.axis_index("c")
        base = wid * b_per_w
        pltpu.sync_copy(px_hbm.at[pl.ds(base, b_per_w)], px_v)
        pltpu.sync_copy(py_hbm.at[pl.ds(base, b_per_w)], py_v)
        copies = []
        for c in range(nchunk):
            ci = jnp.int32(c)
            for j in range(chunk // lanes):
                off = c * chunk + j * lanes
                x = px_v[pl.ds(off, lanes)]
                y = py_v[pl.ds(off, lanes)]
                idx_v[ci, pl.ds(j * lanes, lanes)] = _bitmix_idx(
                    x, y, hash_size - 1)
            cp = pltpu.make_async_copy(
                table_hbm.at[idx_v.at[ci]], rows_v.at[ci], sem)
            cp.start()
            copies.append(cp)
        for cp in copies:
            cp.wait()
        pltpu.sync_copy(rows_v, out_hbm.at[wid])

    return grid_hash_gather


def kernel(positions, table):
    n = positions.shape[0]
    hash_size, dim = table.shape
    px = positions[:, 0]
    py = positions[:, 1]
    out = _build(n, hash_size, dim)(px, py, table)
    return out.reshape(n, dim)


# speedup vs baseline: 6.3718x; 6.3718x over previous
import functools

import jax
import jax.numpy as jnp
from jax import lax
from jax.experimental import pallas as pl
from jax.experimental.pallas import tpu as pltpu
from jax.experimental.pallas import tpu_sc as plsc

_C1 = 2246822507
_C2 = 3266489909


def _u32(v):
    return jnp.uint32(v)


def _mulhi_u32(a, b_const):
    a0 = a & _u32(0xFFFF)
    a1 = a >> _u32(16)
    b0 = _u32(b_const & 0xFFFF)
    b1 = _u32(b_const >> 16)
    t = a0 * b0
    t1 = a1 * b0 + (t >> _u32(16))
    t2 = a0 * b1 + (t1 & _u32(0xFFFF))
    return a1 * b1 + (t1 >> _u32(16)) + (t2 >> _u32(16))


def _floor_i32(x):
    t = x.astype(jnp.int32)
    tf = t.astype(jnp.float32)
    return t - jnp.where(tf > x, jnp.int32(1), jnp.int32(0))


def _bitmix_idx(x, y, mask):
    ix = _floor_i32(x)
    iy = _floor_i32(y)
    ixu = ix.astype(jnp.uint32)
    iyu = iy.astype(jnp.uint32)
    s = (ix >> 31).astype(jnp.uint32)
    d = ixu ^ ((ixu >> _u32(16)) | (s << _u32(16)))
    e_lo = d * _u32(_C1)
    e_hi = _mulhi_u32(d, _C1)
    f_lo = e_lo ^ ((e_lo >> _u32(13)) | (e_hi << _u32(19)))
    f_hi = e_hi ^ (e_hi >> _u32(13))
    t_lo = iyu * _u32(_C2)
    t_hi = _mulhi_u32(iyu, _C2) + jnp.where(
        iy < 0, _u32((1 << 32) - _C2), _u32(0))
    g_lo = f_lo + t_lo
    carry = jnp.where(g_lo < t_lo, _u32(1), _u32(0))
    g_hi = f_hi + t_hi + carry
    out = (g_lo ^ ((g_lo >> _u32(16)) | (g_hi << _u32(16)))) & _u32(mask)
    return out.astype(jnp.int32)


@functools.cache
def _build(n, hash_size, dim):
    assert hash_size & (hash_size - 1) == 0
    info = plsc.get_sparse_core_info()
    nc, ns, lanes = info.num_cores, info.num_subcores, info.num_lanes
    nw = nc * ns
    b_per_w = n // nw
    assert b_per_w * nw == n and lanes == 16 and dim == 64
    ndesc = b_per_w * dim
    chunk = 128
    nchunk = ndesc // chunk
    kgrp = 4
    ngroup = nchunk // kgrp
    vpc = chunk // lanes
    ppc = chunk // dim

    mesh = plsc.VectorSubcoreMesh(core_axis_name="c", subcore_axis_name="s")

    @functools.partial(
        pl.kernel,
        out_type=jax.ShapeDtypeStruct((nw, nchunk, chunk), jnp.float32),
        mesh=mesh,
        compiler_params=pltpu.CompilerParams(
            use_tc_tiling_on_sc=False, needs_layout_passes=False),
        scratch_types=[
            pltpu.VMEM((b_per_w,), jnp.float32),
            pltpu.VMEM((b_per_w,), jnp.float32),
            pltpu.VMEM((b_per_w,), jnp.int32),
            pltpu.VMEM((dim,), jnp.int32),
            pltpu.VMEM((nchunk, chunk), jnp.int32),
            pltpu.VMEM((nchunk, chunk), jnp.float32),
            pltpu.SemaphoreType.DMA,
        ],
    )
    def grid_hash_gather(tflat_hbm, px_hbm, py_hbm, out_hbm,
                         px_v, py_v, b_v, cc_v, idx_v, rows_v, sem0):
        wid = lax.axis_index("s") * nc + lax.axis_index("c")
        base = wid * b_per_w
        pltpu.sync_copy(px_hbm.at[pl.ds(base, b_per_w)], px_v)
        pltpu.sync_copy(py_hbm.at[pl.ds(base, b_per_w)], py_v)

        for j in range(dim // lanes):
            c = jnp.int32(j * lanes) + lax.iota(jnp.int32, lanes)
            cc_v[pl.ds(j * lanes, lanes)] = (
                ((c >> jnp.int32(3)) << jnp.int32(23))
                | ((c & jnp.int32(7)) << jnp.int32(7)))

        for i in range(b_per_w // lanes):
            x = px_v[pl.ds(i * lanes, lanes)]
            y = py_v[pl.ds(i * lanes, lanes)]
            r = _bitmix_idx(x, y, hash_size - 1)
            b_v[pl.ds(i * lanes, lanes)] = (
                ((r >> jnp.int32(7)) << jnp.int32(10))
                | (r & jnp.int32(127)))

        def idx_body(ch):
            for v in range(vpc):
                p = ch * jnp.int32(ppc) + jnp.int32(v // (vpc // ppc))
                bvals = plsc.load_gather(
                    b_v, [jnp.full((lanes,), jnp.int32(0), jnp.int32) + p])
                cc = cc_v[pl.ds((v % (vpc // ppc)) * lanes, lanes)]
                idx_v[ch, pl.ds(v * lanes, lanes)] = bvals + cc
            return ch + jnp.int32(1)

        lax.while_loop(lambda ch: ch < jnp.int32(nchunk), idx_body,
                       jnp.int32(0))

        depth = 4 * kgrp

        def fire(ch):
            pltpu.make_async_copy(
                tflat_hbm.at[idx_v.at[ch]], rows_v.at[ch], sem0).start()

        def wait_one(ch):
            pltpu.make_async_copy(
                tflat_hbm.at[idx_v.at[ch]], rows_v.at[ch], sem0).wait()

        lax.while_loop(lambda ch: ch < jnp.int32(depth),
                       lambda ch: (fire(ch), ch + jnp.int32(1))[1],
                       jnp.int32(0))

        def stream_body(ch):
            wait_one(ch - jnp.int32(depth))
            fire(ch)
            return ch + jnp.int32(1)

        lax.while_loop(lambda ch: ch < jnp.int32(nchunk), stream_body,
                       jnp.int32(depth))

        lax.while_loop(lambda ch: ch < jnp.int32(nchunk),
                       lambda ch: (wait_one(ch), ch + jnp.int32(1))[1],
                       jnp.int32(nchunk - depth))

        pltpu.sync_copy(rows_v, out_hbm.at[wid])

    return grid_hash_gather


def kernel(positions, table):
    n = positions.shape[0]
    hash_size, dim = table.shape
    tflat = table.reshape(hash_size // 128, 128, dim // 8, 8)
    tflat = tflat.transpose(2, 0, 3, 1).reshape(-1)
    px = positions[:, 0]
    py = positions[:, 1]
    out = _build(n, hash_size, dim)(tflat, px, py)
    return out.reshape(n, dim)


# speedup vs baseline: 7.4444x; 1.1683x over previous
import functools

import jax
import jax.numpy as jnp
from jax import lax
from jax.experimental import pallas as pl
from jax.experimental.pallas import tpu as pltpu
from jax.experimental.pallas import tpu_sc as plsc

_C1 = 2246822507
_C2 = 3266489909


def _u32(v):
    return jnp.uint32(v)


def _mulhi_u32(a, b_const):
    a0 = a & _u32(0xFFFF)
    a1 = a >> _u32(16)
    b0 = _u32(b_const & 0xFFFF)
    b1 = _u32(b_const >> 16)
    t = a0 * b0
    t1 = a1 * b0 + (t >> _u32(16))
    t2 = a0 * b1 + (t1 & _u32(0xFFFF))
    return a1 * b1 + (t1 >> _u32(16)) + (t2 >> _u32(16))


def _floor_i32(x):
    t = x.astype(jnp.int32)
    tf = t.astype(jnp.float32)
    return t - jnp.where(tf > x, jnp.int32(1), jnp.int32(0))


def _bitmix_idx(x, y, mask):
    ix = _floor_i32(x)
    iy = _floor_i32(y)
    ixu = ix.astype(jnp.uint32)
    iyu = iy.astype(jnp.uint32)
    s = (ix >> 31).astype(jnp.uint32)
    d = ixu ^ ((ixu >> _u32(16)) | (s << _u32(16)))
    e_lo = d * _u32(_C1)
    e_hi = _mulhi_u32(d, _C1)
    f_lo = e_lo ^ ((e_lo >> _u32(13)) | (e_hi << _u32(19)))
    f_hi = e_hi ^ (e_hi >> _u32(13))
    t_lo = iyu * _u32(_C2)
    t_hi = _mulhi_u32(iyu, _C2) + jnp.where(
        iy < 0, _u32((1 << 32) - _C2), _u32(0))
    g_lo = f_lo + t_lo
    carry = jnp.where(g_lo < t_lo, _u32(1), _u32(0))
    g_hi = f_hi + t_hi + carry
    out = (g_lo ^ ((g_lo >> _u32(16)) | (g_hi << _u32(16)))) & _u32(mask)
    return out.astype(jnp.int32)


@functools.cache
def _build(n, hash_size, dim):
    assert hash_size & (hash_size - 1) == 0
    info = plsc.get_sparse_core_info()
    nc, ns, lanes = info.num_cores, info.num_subcores, info.num_lanes
    nw = nc * ns
    b_per_w = n // nw
    assert b_per_w * nw == n and lanes == 16 and dim == 64 and b_per_w == 512
    nchunk = b_per_w * dim // 128
    trw = b_per_w // 128
    tcb = dim // 8

    mesh = plsc.VectorSubcoreMesh(core_axis_name="c", subcore_axis_name="s")

    @functools.partial(
        pl.kernel,
        out_type=jax.ShapeDtypeStruct((tcb, nw, trw * 8, 128), jnp.float32),
        mesh=mesh,
        compiler_params=pltpu.CompilerParams(
            use_tc_tiling_on_sc=False, needs_layout_passes=False),
        scratch_types=[
            pltpu.VMEM((b_per_w,), jnp.float32),
            pltpu.VMEM((b_per_w,), jnp.float32),
            pltpu.VMEM((b_per_w,), jnp.int32),
            pltpu.VMEM((nchunk, 128), jnp.int32),
            pltpu.VMEM((nchunk, 128), jnp.float32),
            pltpu.SemaphoreType.DMA,
        ],
    )
    def grid_hash_gather(tflat_hbm, px_hbm, py_hbm, out_hbm,
                         px_v, py_v, b_v, idx_v, rows_v, sem0):
        wid = lax.axis_index("s") * nc + lax.axis_index("c")
        base = wid * b_per_w
        pltpu.sync_copy(px_hbm.at[pl.ds(base, b_per_w)], px_v)
        pltpu.sync_copy(py_hbm.at[pl.ds(base, b_per_w)], py_v)

        for i in range(b_per_w // lanes):
            x = px_v[pl.ds(i * lanes, lanes)]
            y = py_v[pl.ds(i * lanes, lanes)]
            r = _bitmix_idx(x, y, hash_size - 1)
            b_v[pl.ds(i * lanes, lanes)] = (
                ((r >> jnp.int32(7)) << jnp.int32(10))
                | (r & jnp.int32(127)))

        def idx_body(k):
            ci = k & jnp.int32(7)
            tr_loc = (k >> jnp.int32(3)) & jnp.int32(trw - 1)
            tc = k >> jnp.int32(5)
            cc = (tc << jnp.int32(23)) | (ci << jnp.int32(7))
            ccv = jnp.full((lanes,), jnp.int32(0), jnp.int32) + cc
            pbase = tr_loc * jnp.int32(128)
            for u in range(128 // lanes):
                bv = b_v[pl.ds(pbase + jnp.int32(u * lanes), lanes)]
                idx_v[k, pl.ds(u * lanes, lanes)] = bv + ccv
            return k + jnp.int32(1)

        lax.while_loop(lambda k: k < jnp.int32(nchunk), idx_body,
                       jnp.int32(0))

        depth = 16

        def fire(ch):
            pltpu.make_async_copy(
                tflat_hbm.at[idx_v.at[ch]], rows_v.at[ch], sem0).start()

        def wait_one(ch):
            pltpu.make_async_copy(
                tflat_hbm.at[idx_v.at[ch]], rows_v.at[ch], sem0).wait()

        lax.while_loop(lambda ch: ch < jnp.int32(depth),
                       lambda ch: (fire(ch), ch + jnp.int32(1))[1],
                       jnp.int32(0))

        def stream_body(ch):
            wait_one(ch - jnp.int32(depth))
            fire(ch)
            return ch + jnp.int32(1)

        lax.while_loop(lambda ch: ch < jnp.int32(nchunk), stream_body,
                       jnp.int32(depth))

        lax.while_loop(lambda ch: ch < jnp.int32(nchunk),
                       lambda ch: (wait_one(ch), ch + jnp.int32(1))[1],
                       jnp.int32(nchunk - depth))

        for t in range(tcb):
            pltpu.sync_copy(rows_v.at[pl.ds(t * (nchunk // tcb),
                                            nchunk // tcb)],
                            out_hbm.at[jnp.int32(t), wid])

    return grid_hash_gather


def kernel(positions, table):
    n = positions.shape[0]
    hash_size, dim = table.shape
    tflat = table.reshape(hash_size // 128, 128, dim // 8, 8)
    tflat = tflat.transpose(2, 0, 3, 1).reshape(-1)
    px = positions[:, 0]
    py = positions[:, 1]
    out = _build(n, hash_size, dim)(tflat, px, py)
    tcb = dim // 8
    out = out.reshape(tcb, n // 512, 4, 8, 128)
    out = out.transpose(1, 2, 4, 0, 3).reshape(n, dim)
    return out


# speedup vs baseline: 8.1986x; 1.1013x over previous
import functools

import jax
import jax.numpy as jnp
from jax import lax
from jax.experimental import pallas as pl
from jax.experimental.pallas import tpu as pltpu
from jax.experimental.pallas import tpu_sc as plsc

_C1 = 2246822507
_C2 = 3266489909


def _u32(v):
    return jnp.uint32(v)


def _mulhi_u32(a, b_const):
    a0 = a & _u32(0xFFFF)
    a1 = a >> _u32(16)
    b0 = _u32(b_const & 0xFFFF)
    b1 = _u32(b_const >> 16)
    t = a0 * b0
    t1 = a1 * b0 + (t >> _u32(16))
    t2 = a0 * b1 + (t1 & _u32(0xFFFF))
    return a1 * b1 + (t1 >> _u32(16)) + (t2 >> _u32(16))


def _floor_i32(x):
    t = x.astype(jnp.int32)
    tf = t.astype(jnp.float32)
    return t - jnp.where(tf > x, jnp.int32(1), jnp.int32(0))


def _bitmix_idx(x, y, mask):
    ix = _floor_i32(x)
    iy = _floor_i32(y)
    ixu = ix.astype(jnp.uint32)
    iyu = iy.astype(jnp.uint32)
    s = (ix >> 31).astype(jnp.uint32)
    d = ixu ^ ((ixu >> _u32(16)) | (s << _u32(16)))
    e_lo = d * _u32(_C1)
    e_hi = _mulhi_u32(d, _C1)
    f_lo = e_lo ^ ((e_lo >> _u32(13)) | (e_hi << _u32(19)))
    f_hi = e_hi ^ (e_hi >> _u32(13))
    t_lo = iyu * _u32(_C2)
    t_hi = _mulhi_u32(iyu, _C2) + jnp.where(
        iy < 0, _u32((1 << 32) - _C2), _u32(0))
    g_lo = f_lo + t_lo
    carry = jnp.where(g_lo < t_lo, _u32(1), _u32(0))
    g_hi = f_hi + t_hi + carry
    out = (g_lo ^ ((g_lo >> _u32(16)) | (g_hi << _u32(16)))) & _u32(mask)
    return out.astype(jnp.int32)


@functools.cache
def _build(n, hash_size, dim):
    assert hash_size & (hash_size - 1) == 0
    info = plsc.get_sparse_core_info()
    nc, ns, lanes = info.num_cores, info.num_subcores, info.num_lanes
    nw = nc * ns
    b_per_w = n // nw
    assert b_per_w * nw == n and lanes == 16 and dim == 64 and b_per_w == 512
    nchunk = b_per_w * dim // 128
    trw = b_per_w // 128
    tcb = dim // 8

    mesh = plsc.VectorSubcoreMesh(core_axis_name="c", subcore_axis_name="s")

    @functools.partial(
        pl.kernel,
        out_type=jax.ShapeDtypeStruct((tcb, nw, trw * 8, 128), jnp.float32),
        mesh=mesh,
        compiler_params=pltpu.CompilerParams(
            use_tc_tiling_on_sc=False, needs_layout_passes=False),
        scratch_types=[
            pltpu.VMEM((b_per_w,), jnp.float32),
            pltpu.VMEM((b_per_w,), jnp.float32),
            pltpu.VMEM((b_per_w,), jnp.int32),
            pltpu.VMEM((nchunk, 128), jnp.int32),
            pltpu.VMEM((nchunk, 128), jnp.float32),
            pltpu.SemaphoreType.DMA,
        ],
    )
    def grid_hash_gather(tflat_hbm, px_hbm, py_hbm, out_hbm,
                         px_v, py_v, b_v, idx_v, rows_v, sem0):
        wid = lax.axis_index("s") * nc + lax.axis_index("c")
        base = wid * b_per_w
        pltpu.sync_copy(px_hbm.at[pl.ds(base, b_per_w)], px_v)
        pltpu.sync_copy(py_hbm.at[pl.ds(base, b_per_w)], py_v)

        for i in range(b_per_w // lanes):
            x = px_v[pl.ds(i * lanes, lanes)]
            y = py_v[pl.ds(i * lanes, lanes)]
            r = _bitmix_idx(x, y, hash_size - 1)
            b_v[pl.ds(i * lanes, lanes)] = (
                ((r >> jnp.int32(7)) << jnp.int32(10))
                | (r & jnp.int32(127)))

        def compute_idx(k):
            ci = k & jnp.int32(7)
            tr_loc = (k >> jnp.int32(3)) & jnp.int32(trw - 1)
            tc = k >> jnp.int32(5)
            cc = (tc << jnp.int32(23)) | (ci << jnp.int32(7))
            ccv = jnp.full((lanes,), jnp.int32(0), jnp.int32) + cc
            pbase = tr_loc * jnp.int32(128)
            for u in range(128 // lanes):
                bv = b_v[pl.ds(pbase + jnp.int32(u * lanes), lanes)]
                idx_v[k, pl.ds(u * lanes, lanes)] = bv + ccv

        def fire(ch):
            pltpu.make_async_copy(
                tflat_hbm.at[idx_v.at[ch]], rows_v.at[ch], sem0).start()

        def wait_one(ch):
            pltpu.make_async_copy(
                tflat_hbm.at[idx_v.at[ch]], rows_v.at[ch], sem0).wait()

        depth = 16

        def prime_body(ch):
            compute_idx(ch)
            fire(ch)
            return ch + jnp.int32(1)

        lax.while_loop(lambda ch: ch < jnp.int32(depth), prime_body,
                       jnp.int32(0))

        def stream_body(ch):
            compute_idx(ch)
            wait_one(ch - jnp.int32(depth))
            fire(ch)
            return ch + jnp.int32(1)

        lax.while_loop(lambda ch: ch < jnp.int32(nchunk), stream_body,
                       jnp.int32(depth))

        lax.while_loop(lambda ch: ch < jnp.int32(nchunk),
                       lambda ch: (wait_one(ch), ch + jnp.int32(1))[1],
                       jnp.int32(nchunk - depth))

        for t in range(tcb):
            pltpu.sync_copy(rows_v.at[pl.ds(t * (nchunk // tcb),
                                            nchunk // tcb)],
                            out_hbm.at[jnp.int32(t), wid])

    return grid_hash_gather


def kernel(positions, table):
    n = positions.shape[0]
    hash_size, dim = table.shape
    tflat = table.reshape(hash_size // 128, 128, dim // 8, 8)
    tflat = tflat.transpose(2, 0, 3, 1).reshape(-1)
    px = positions[:, 0]
    py = positions[:, 1]
    out = _build(n, hash_size, dim)(tflat, px, py)
    tcb = dim // 8
    out = out.reshape(tcb, n // 512, 4, 8, 128)
    out = out.transpose(1, 2, 4, 0, 3).reshape(n, dim)
    return out


# speedup vs baseline: 8.6157x; 1.0509x over previous
import functools

import jax
import jax.numpy as jnp
from jax import lax
from jax.experimental import pallas as pl
from jax.experimental.pallas import tpu as pltpu
from jax.experimental.pallas import tpu_sc as plsc

_C1 = 2246822507
_C2 = 3266489909


def _u32(v):
    return jnp.uint32(v)


def _mulhi_u32(a, b_const):
    a0 = a & _u32(0xFFFF)
    a1 = a >> _u32(16)
    b0 = _u32(b_const & 0xFFFF)
    b1 = _u32(b_const >> 16)
    t = a0 * b0
    t1 = a1 * b0 + (t >> _u32(16))
    t2 = a0 * b1 + (t1 & _u32(0xFFFF))
    return a1 * b1 + (t1 >> _u32(16)) + (t2 >> _u32(16))


def _floor_i32(x):
    t = x.astype(jnp.int32)
    tf = t.astype(jnp.float32)
    return t - jnp.where(tf > x, jnp.int32(1), jnp.int32(0))


def _bitmix_idx(x, y, mask):
    ix = _floor_i32(x)
    iy = _floor_i32(y)
    ixu = ix.astype(jnp.uint32)
    iyu = iy.astype(jnp.uint32)
    s = (ix >> 31).astype(jnp.uint32)
    d = ixu ^ ((ixu >> _u32(16)) | (s << _u32(16)))
    e_lo = d * _u32(_C1)
    e_hi = _mulhi_u32(d, _C1)
    f_lo = e_lo ^ ((e_lo >> _u32(13)) | (e_hi << _u32(19)))
    f_hi = e_hi ^ (e_hi >> _u32(13))
    t_lo = iyu * _u32(_C2)
    t_hi = _mulhi_u32(iyu, _C2) + jnp.where(
        iy < 0, _u32((1 << 32) - _C2), _u32(0))
    g_lo = f_lo + t_lo
    carry = jnp.where(g_lo < t_lo, _u32(1), _u32(0))
    g_hi = f_hi + t_hi + carry
    out = (g_lo ^ ((g_lo >> _u32(16)) | (g_hi << _u32(16)))) & _u32(mask)
    return out.astype(jnp.int32)


@functools.cache
def _build(n, hash_size, dim):
    assert hash_size & (hash_size - 1) == 0
    info = plsc.get_sparse_core_info()
    nc, ns, lanes = info.num_cores, info.num_subcores, info.num_lanes
    nw = nc * ns
    b_per_w = n // nw
    assert b_per_w * nw == n and lanes == 16 and dim == 64 and b_per_w == 512
    nchunk = b_per_w * dim // 128
    trw = b_per_w // 128
    tcb = dim // 8

    mesh = plsc.VectorSubcoreMesh(core_axis_name="c", subcore_axis_name="s")

    @functools.partial(
        pl.kernel,
        out_type=jax.ShapeDtypeStruct((tcb, nw, trw * 8, 128), jnp.float32),
        mesh=mesh,
        compiler_params=pltpu.CompilerParams(
            use_tc_tiling_on_sc=False, needs_layout_passes=False),
        scratch_types=[
            pltpu.VMEM((b_per_w,), jnp.float32),
            pltpu.VMEM((b_per_w,), jnp.float32),
            pltpu.VMEM((b_per_w,), jnp.int32),
            pltpu.VMEM((nchunk, 128), jnp.int32),
            pltpu.VMEM((nchunk, 128), jnp.float32),
            pltpu.SemaphoreType.DMA,
        ],
    )
    def grid_hash_gather(tflat_hbm, px_hbm, py_hbm, out_hbm,
                         px_v, py_v, b_v, idx_v, rows_v, sem0):
        wid = lax.axis_index("s") * nc + lax.axis_index("c")
        base = wid * b_per_w
        pltpu.sync_copy(px_hbm.at[pl.ds(base, b_per_w)], px_v)
        pltpu.sync_copy(py_hbm.at[pl.ds(base, b_per_w)], py_v)

        for i in range(b_per_w // lanes):
            x = px_v[pl.ds(i * lanes, lanes)]
            y = py_v[pl.ds(i * lanes, lanes)]
            r = _bitmix_idx(x, y, hash_size - 1)
            b_v[pl.ds(i * lanes, lanes)] = (
                ((r >> jnp.int32(7)) << jnp.int32(10))
                | (r & jnp.int32(127)))

        def compute_idx(k):
            ci = k & jnp.int32(7)
            tr_loc = (k >> jnp.int32(3)) & jnp.int32(trw - 1)
            tc = k >> jnp.int32(5)
            cc = (tc << jnp.int32(23)) | (ci << jnp.int32(7))
            ccv = jnp.full((lanes,), jnp.int32(0), jnp.int32) + cc
            pbase = tr_loc * jnp.int32(128)
            for u in range(128 // lanes):
                bv = b_v[pl.ds(pbase + jnp.int32(u * lanes), lanes)]
                idx_v[k, pl.ds(u * lanes, lanes)] = bv + ccv

        def fire(ch):
            pltpu.make_async_copy(
                tflat_hbm.at[idx_v.at[ch]], rows_v.at[ch], sem0).start()

        def wait_one(ch):
            pltpu.make_async_copy(
                tflat_hbm.at[idx_v.at[ch]], rows_v.at[ch], sem0).wait()

        depth = 32

        def prime_body(ch):
            compute_idx(ch)
            fire(ch)
            return ch + jnp.int32(1)

        lax.while_loop(lambda ch: ch < jnp.int32(depth), prime_body,
                       jnp.int32(0))

        def stream_body(ch):
            for u in range(2):
                c = ch + jnp.int32(u)
                compute_idx(c)
                wait_one(c - jnp.int32(depth))
                fire(c)
            return ch + jnp.int32(2)

        lax.while_loop(lambda ch: ch < jnp.int32(nchunk), stream_body,
                       jnp.int32(depth))

        lax.while_loop(lambda ch: ch < jnp.int32(nchunk),
                       lambda ch: (wait_one(ch), ch + jnp.int32(1))[1],
                       jnp.int32(nchunk - depth))

        for t in range(tcb):
            pltpu.sync_copy(rows_v.at[pl.ds(t * (nchunk // tcb),
                                            nchunk // tcb)],
                            out_hbm.at[jnp.int32(t), wid])

    return grid_hash_gather


def kernel(positions, table):
    n = positions.shape[0]
    hash_size, dim = table.shape
    tflat = table.reshape(hash_size // 128, 128, dim // 8, 8)
    tflat = tflat.transpose(2, 0, 3, 1).reshape(-1)
    px = positions[:, 0]
    py = positions[:, 1]
    out = _build(n, hash_size, dim)(tflat, px, py)
    tcb = dim // 8
    out = out.reshape(tcb, n // 512, 4, 8, 128)
    out = out.transpose(1, 2, 4, 0, 3).reshape(n, dim)
    return out


# speedup vs baseline: 9.2270x; 1.0710x over previous
import functools

import jax
import jax.numpy as jnp
from jax import lax
from jax.experimental import pallas as pl
from jax.experimental.pallas import tpu as pltpu
from jax.experimental.pallas import tpu_sc as plsc

_C1 = 2246822507
_C2 = 3266489909


def _u32(v):
    return jnp.uint32(v)


def _mulhi_u32(a, b_const):
    a0 = a & _u32(0xFFFF)
    a1 = a >> _u32(16)
    b0 = _u32(b_const & 0xFFFF)
    b1 = _u32(b_const >> 16)
    t = a0 * b0
    t1 = a1 * b0 + (t >> _u32(16))
    t2 = a0 * b1 + (t1 & _u32(0xFFFF))
    return a1 * b1 + (t1 >> _u32(16)) + (t2 >> _u32(16))


def _floor_i32(x):
    t = x.astype(jnp.int32)
    tf = t.astype(jnp.float32)
    return t - jnp.where(tf > x, jnp.int32(1), jnp.int32(0))


def _bitmix_idx(x, y, mask):
    ix = _floor_i32(x)
    iy = _floor_i32(y)
    ixu = ix.astype(jnp.uint32)
    iyu = iy.astype(jnp.uint32)
    s = (ix >> 31).astype(jnp.uint32)
    d = ixu ^ ((ixu >> _u32(16)) | (s << _u32(16)))
    e_lo = d * _u32(_C1)
    e_hi = _mulhi_u32(d, _C1)
    f_lo = e_lo ^ ((e_lo >> _u32(13)) | (e_hi << _u32(19)))
    f_hi = e_hi ^ (e_hi >> _u32(13))
    t_lo = iyu * _u32(_C2)
    t_hi = _mulhi_u32(iyu, _C2) + jnp.where(
        iy < 0, _u32((1 << 32) - _C2), _u32(0))
    g_lo = f_lo + t_lo
    carry = jnp.where(g_lo < t_lo, _u32(1), _u32(0))
    g_hi = f_hi + t_hi + carry
    out = (g_lo ^ ((g_lo >> _u32(16)) | (g_hi << _u32(16)))) & _u32(mask)
    return out.astype(jnp.int32)


@functools.cache
def _build(n, hash_size, dim):
    assert hash_size & (hash_size - 1) == 0
    info = plsc.get_sparse_core_info()
    nc, ns, lanes = info.num_cores, info.num_subcores, info.num_lanes
    nw = nc * ns
    b_per_w = n // nw
    assert b_per_w * nw == n and lanes == 16 and dim == 64 and b_per_w == 512
    nchunk = b_per_w * dim // 128
    trw = b_per_w // 128
    tcb = dim // 8

    mesh = plsc.VectorSubcoreMesh(core_axis_name="c", subcore_axis_name="s")

    @functools.partial(
        pl.kernel,
        out_type=jax.ShapeDtypeStruct((tcb, nw, trw * 8, 128), jnp.float32),
        mesh=mesh,
        compiler_params=pltpu.CompilerParams(
            use_tc_tiling_on_sc=False, needs_layout_passes=False),
        scratch_types=[
            pltpu.VMEM((b_per_w,), jnp.float32),
            pltpu.VMEM((b_per_w,), jnp.float32),
            pltpu.VMEM((b_per_w,), jnp.int32),
            pltpu.VMEM((nchunk, 128), jnp.int32),
            pltpu.VMEM((nchunk, 128), jnp.float32),
            pltpu.SemaphoreType.DMA,
        ],
    )
    def grid_hash_gather(tflat_hbm, px_hbm, py_hbm, out_hbm,
                         px_v, py_v, b_v, idx_v, rows_v, sem0):
        wid = lax.axis_index("s") * nc + lax.axis_index("c")
        base = wid * b_per_w
        pltpu.sync_copy(px_hbm.at[pl.ds(base, b_per_w)], px_v)
        pltpu.sync_copy(py_hbm.at[pl.ds(base, b_per_w)], py_v)

        for i in range(b_per_w // lanes):
            x = px_v[pl.ds(i * lanes, lanes)]
            y = py_v[pl.ds(i * lanes, lanes)]
            r = _bitmix_idx(x, y, hash_size - 1)
            b_v[pl.ds(i * lanes, lanes)] = (
                ((r >> jnp.int32(7)) << jnp.int32(10))
                | (r & jnp.int32(127)))

        def compute_idx(k):
            ci = k & jnp.int32(7)
            tr_loc = (k >> jnp.int32(3)) & jnp.int32(trw - 1)
            tc = k >> jnp.int32(5)
            cc = (tc << jnp.int32(23)) | (ci << jnp.int32(7))
            ccv = jnp.full((lanes,), jnp.int32(0), jnp.int32) + cc
            pbase = tr_loc * jnp.int32(128)
            for u in range(128 // lanes):
                bv = b_v[pl.ds(pbase + jnp.int32(u * lanes), lanes)]
                idx_v[k, pl.ds(u * lanes, lanes)] = bv + ccv

        def fire(ch):
            pltpu.make_async_copy(
                tflat_hbm.at[idx_v.at[ch]], rows_v.at[ch], sem0).start()

        def wait_one(ch):
            pltpu.make_async_copy(
                px_hbm.at[pl.ds(jnp.int32(0), 128)], rows_v.at[ch],
                sem0).wait()

        depth = 64

        def prime_body(ch):
            compute_idx(ch)
            fire(ch)
            return ch + jnp.int32(1)

        lax.while_loop(lambda ch: ch < jnp.int32(depth), prime_body,
                       jnp.int32(0))

        def stream_body(ch):
            for u in range(2):
                c = ch + jnp.int32(u)
                compute_idx(c)
                wait_one(c - jnp.int32(depth))
                fire(c)
            return ch + jnp.int32(2)

        lax.while_loop(lambda ch: ch < jnp.int32(nchunk), stream_body,
                       jnp.int32(depth))

        lax.while_loop(lambda ch: ch < jnp.int32(nchunk),
                       lambda ch: (wait_one(ch), ch + jnp.int32(1))[1],
                       jnp.int32(nchunk - depth))

        for t in range(tcb):
            pltpu.sync_copy(rows_v.at[pl.ds(t * (nchunk // tcb),
                                            nchunk // tcb)],
                            out_hbm.at[jnp.int32(t), wid])

    return grid_hash_gather


def kernel(positions, table):
    n = positions.shape[0]
    hash_size, dim = table.shape
    tflat = table.reshape(hash_size // 128, 128, dim // 8, 8)
    tflat = tflat.transpose(2, 0, 3, 1).reshape(-1)
    px = positions[:, 0]
    py = positions[:, 1]
    out = _build(n, hash_size, dim)(tflat, px, py)
    tcb = dim // 8
    out = out.reshape(tcb, n // 512, 4, 8, 128)
    out = out.transpose(1, 2, 4, 0, 3).reshape(n, dim)
    return out


# speedup vs baseline: 9.3988x; 1.0186x over previous
import functools

import jax
import jax.numpy as jnp
from jax import lax
from jax.experimental import pallas as pl
from jax.experimental.pallas import tpu as pltpu
from jax.experimental.pallas import tpu_sc as plsc

_C1 = 2246822507
_C2 = 3266489909


def _u32(v):
    return jnp.uint32(v)


def _mulhi_u32(a, b_const):
    a0 = a & _u32(0xFFFF)
    a1 = a >> _u32(16)
    b0 = _u32(b_const & 0xFFFF)
    b1 = _u32(b_const >> 16)
    t = a0 * b0
    t1 = a1 * b0 + (t >> _u32(16))
    t2 = a0 * b1 + (t1 & _u32(0xFFFF))
    return a1 * b1 + (t1 >> _u32(16)) + (t2 >> _u32(16))


def _floor_i32(x):
    t = x.astype(jnp.int32)
    tf = t.astype(jnp.float32)
    return t - jnp.where(tf > x, jnp.int32(1), jnp.int32(0))


def _bitmix_idx(x, y, mask):
    ix = _floor_i32(x)
    iy = _floor_i32(y)
    ixu = ix.astype(jnp.uint32)
    iyu = iy.astype(jnp.uint32)
    s = (ix >> 31).astype(jnp.uint32)
    d = ixu ^ ((ixu >> _u32(16)) | (s << _u32(16)))
    e_lo = d * _u32(_C1)
    e_hi = _mulhi_u32(d, _C1)
    f_lo = e_lo ^ ((e_lo >> _u32(13)) | (e_hi << _u32(19)))
    f_hi = e_hi ^ (e_hi >> _u32(13))
    t_lo = iyu * _u32(_C2)
    t_hi = _mulhi_u32(iyu, _C2) + jnp.where(
        iy < 0, _u32((1 << 32) - _C2), _u32(0))
    g_lo = f_lo + t_lo
    carry = jnp.where(g_lo < t_lo, _u32(1), _u32(0))
    g_hi = f_hi + t_hi + carry
    out = (g_lo ^ ((g_lo >> _u32(16)) | (g_hi << _u32(16)))) & _u32(mask)
    return out.astype(jnp.int32)


@functools.cache
def _build(n, hash_size, dim):
    assert hash_size & (hash_size - 1) == 0
    info = plsc.get_sparse_core_info()
    nc, ns, lanes = info.num_cores, info.num_subcores, info.num_lanes
    nw = nc * ns
    b_per_w = n // nw
    assert b_per_w * nw == n and lanes == 16 and dim == 64 and b_per_w == 512
    nchunk = b_per_w * dim // 128
    trw = b_per_w // 128
    tcb = dim // 8

    mesh = plsc.VectorSubcoreMesh(core_axis_name="c", subcore_axis_name="s")

    @functools.partial(
        pl.kernel,
        out_type=jax.ShapeDtypeStruct((tcb, nw, trw * 8, 128), jnp.float32),
        mesh=mesh,
        compiler_params=pltpu.CompilerParams(
            use_tc_tiling_on_sc=False, needs_layout_passes=False),
        scratch_types=[
            pltpu.VMEM((b_per_w,), jnp.float32),
            pltpu.VMEM((b_per_w,), jnp.float32),
            pltpu.VMEM((b_per_w,), jnp.int32),
            pltpu.VMEM((nchunk, 128), jnp.int32),
            pltpu.VMEM((nchunk, 128), jnp.float32),
            pltpu.VMEM((512,), jnp.float32),
            pltpu.SemaphoreType.DMA,
        ],
    )
    def grid_hash_gather(tflat_hbm, px_hbm, py_hbm, out_hbm,
                         px_v, py_v, b_v, idx_v, rows_v, drain_v, sem0):
        wid = lax.axis_index("s") * nc + lax.axis_index("c")
        base = wid * b_per_w
        pltpu.sync_copy(px_hbm.at[pl.ds(base, b_per_w)], px_v)
        pltpu.sync_copy(py_hbm.at[pl.ds(base, b_per_w)], py_v)

        for i in range(b_per_w // lanes):
            x = px_v[pl.ds(i * lanes, lanes)]
            y = py_v[pl.ds(i * lanes, lanes)]
            r = _bitmix_idx(x, y, hash_size - 1)
            b_v[pl.ds(i * lanes, lanes)] = (
                ((r >> jnp.int32(7)) << jnp.int32(10))
                | (r & jnp.int32(127)))

        def compute_idx(k):
            ci = k & jnp.int32(7)
            tr_loc = (k >> jnp.int32(3)) & jnp.int32(trw - 1)
            tc = k >> jnp.int32(5)
            cc = (tc << jnp.int32(23)) | (ci << jnp.int32(7))
            ccv = jnp.full((lanes,), jnp.int32(0), jnp.int32) + cc
            pbase = tr_loc * jnp.int32(128)
            for u in range(128 // lanes):
                bv = b_v[pl.ds(pbase + jnp.int32(u * lanes), lanes)]
                idx_v[k, pl.ds(u * lanes, lanes)] = bv + ccv

        def fire(ch):
            pltpu.make_async_copy(
                tflat_hbm.at[idx_v.at[ch]], rows_v.at[ch], sem0).start()

        def wait_four():
            pltpu.make_async_copy(
                px_hbm.at[pl.ds(jnp.int32(0), 512)], drain_v, sem0).wait()

        depth = 64

        def prime_body(ch):
            compute_idx(ch)
            fire(ch)
            return ch + jnp.int32(1)

        lax.while_loop(lambda ch: ch < jnp.int32(depth), prime_body,
                       jnp.int32(0))

        def stream_body(ch):
            for u in range(4):
                c = ch + jnp.int32(u)
                compute_idx(c)
                fire(c)
            wait_four()
            return ch + jnp.int32(4)

        lax.while_loop(lambda ch: ch < jnp.int32(nchunk), stream_body,
                       jnp.int32(depth))

        lax.while_loop(lambda ch: ch < jnp.int32(depth // 4),
                       lambda ch: (wait_four(), ch + jnp.int32(1))[1],
                       jnp.int32(0))

        for t in range(tcb):
            pltpu.sync_copy(rows_v.at[pl.ds(t * (nchunk // tcb),
                                            nchunk // tcb)],
                            out_hbm.at[jnp.int32(t), wid])

    return grid_hash_gather


def kernel(positions, table):
    n = positions.shape[0]
    hash_size, dim = table.shape
    tflat = table.reshape(hash_size // 128, 128, dim // 8, 8)
    tflat = tflat.transpose(2, 0, 3, 1).reshape(-1)
    px = positions[:, 0]
    py = positions[:, 1]
    out = _build(n, hash_size, dim)(tflat, px, py)
    tcb = dim // 8
    out = out.reshape(tcb, n // 512, 4, 8, 128)
    out = out.transpose(1, 2, 4, 0, 3).reshape(n, dim)
    return out


# speedup vs baseline: 9.8958x; 1.0529x over previous
import functools

import jax
import jax.numpy as jnp
from jax import lax
from jax.experimental import pallas as pl
from jax.experimental.pallas import tpu as pltpu
from jax.experimental.pallas import tpu_sc as plsc

_C1 = 2246822507
_C2 = 3266489909


def _u32(v):
    return jnp.uint32(v)


def _mulhi_u32(a, b_const):
    a0 = a & _u32(0xFFFF)
    a1 = a >> _u32(16)
    b0 = _u32(b_const & 0xFFFF)
    b1 = _u32(b_const >> 16)
    t = a0 * b0
    t1 = a1 * b0 + (t >> _u32(16))
    t2 = a0 * b1 + (t1 & _u32(0xFFFF))
    return a1 * b1 + (t1 >> _u32(16)) + (t2 >> _u32(16))


def _floor_i32(x):
    t = x.astype(jnp.int32)
    tf = t.astype(jnp.float32)
    return t - jnp.where(tf > x, jnp.int32(1), jnp.int32(0))


def _bitmix_idx(x, y, mask):
    ix = _floor_i32(x)
    iy = _floor_i32(y)
    ixu = ix.astype(jnp.uint32)
    iyu = iy.astype(jnp.uint32)
    s = (ix >> 31).astype(jnp.uint32)
    d = ixu ^ ((ixu >> _u32(16)) | (s << _u32(16)))
    e_lo = d * _u32(_C1)
    e_hi = _mulhi_u32(d, _C1)
    f_lo = e_lo ^ ((e_lo >> _u32(13)) | (e_hi << _u32(19)))
    f_hi = e_hi ^ (e_hi >> _u32(13))
    t_lo = iyu * _u32(_C2)
    t_hi = _mulhi_u32(iyu, _C2) + jnp.where(
        iy < 0, _u32((1 << 32) - _C2), _u32(0))
    g_lo = f_lo + t_lo
    carry = jnp.where(g_lo < t_lo, _u32(1), _u32(0))
    g_hi = f_hi + t_hi + carry
    out = (g_lo ^ ((g_lo >> _u32(16)) | (g_hi << _u32(16)))) & _u32(mask)
    return out.astype(jnp.int32)


@functools.cache
def _build(n, hash_size, dim):
    assert hash_size & (hash_size - 1) == 0
    info = plsc.get_sparse_core_info()
    nc, ns, lanes = info.num_cores, info.num_subcores, info.num_lanes
    nw = nc * ns
    b_per_w = n // nw
    assert b_per_w * nw == n and lanes == 16 and dim == 64 and b_per_w == 512
    nchunk = b_per_w * dim // 128
    trw = b_per_w // 128
    tcb = dim // 8

    mesh = plsc.VectorSubcoreMesh(core_axis_name="c", subcore_axis_name="s")

    @functools.partial(
        pl.kernel,
        out_type=jax.ShapeDtypeStruct((tcb, nw, trw * 8, 128), jnp.float32),
        mesh=mesh,
        compiler_params=pltpu.CompilerParams(
            use_tc_tiling_on_sc=False, needs_layout_passes=False),
        scratch_types=[
            pltpu.VMEM((b_per_w,), jnp.float32),
            pltpu.VMEM((b_per_w,), jnp.float32),
            pltpu.VMEM((b_per_w,), jnp.int32),
            pltpu.VMEM((nchunk, 128), jnp.int32),
            pltpu.VMEM((nchunk, 128), jnp.float32),
            pltpu.VMEM((512,), jnp.float32),
            pltpu.SemaphoreType.DMA,
        ],
    )
    def grid_hash_gather(tflat_hbm, px_hbm, py_hbm, out_hbm,
                         px_v, py_v, b_v, idx_v, rows_v, drain_v, sem0):
        wid = lax.axis_index("s") * nc + lax.axis_index("c")
        base = wid * b_per_w
        pltpu.sync_copy(px_hbm.at[pl.ds(base, b_per_w)], px_v)
        pltpu.sync_copy(py_hbm.at[pl.ds(base, b_per_w)], py_v)

        for i in range(b_per_w // lanes):
            x = px_v[pl.ds(i * lanes, lanes)]
            y = py_v[pl.ds(i * lanes, lanes)]
            r = _bitmix_idx(x, y, hash_size - 1)
            b_v[pl.ds(i * lanes, lanes)] = (
                ((r >> jnp.int32(7)) << jnp.int32(10))
                | (r & jnp.int32(127)))

        def compute_idx(k):
            ci = k & jnp.int32(7)
            tr_loc = (k >> jnp.int32(3)) & jnp.int32(trw - 1)
            tc = k >> jnp.int32(5)
            cc = (tc << jnp.int32(23)) | (ci << jnp.int32(7))
            ccv = jnp.full((lanes,), jnp.int32(0), jnp.int32) + cc
            pbase = tr_loc * jnp.int32(128)
            for u in range(128 // lanes):
                bv = b_v[pl.ds(pbase + jnp.int32(u * lanes), lanes)]
                idx_v[k, pl.ds(u * lanes, lanes)] = bv + ccv

        def fire(ch):
            pltpu.make_async_copy(
                tflat_hbm.at[idx_v.at[ch]], rows_v.at[ch], sem0).start()

        def wait_four():
            pltpu.make_async_copy(
                px_hbm.at[pl.ds(jnp.int32(0), 512)], drain_v, sem0).wait()

        depth = 128

        def prime_body(ch):
            compute_idx(ch)
            fire(ch)
            return ch + jnp.int32(1)

        lax.while_loop(lambda ch: ch < jnp.int32(depth), prime_body,
                       jnp.int32(0))

        def stream_body(ch):
            for u in range(4):
                c = ch + jnp.int32(u)
                compute_idx(c)
                fire(c)
            wait_four()
            return ch + jnp.int32(4)

        lax.while_loop(lambda ch: ch < jnp.int32(nchunk), stream_body,
                       jnp.int32(depth))

        lax.while_loop(lambda ch: ch < jnp.int32(depth // 4),
                       lambda ch: (wait_four(), ch + jnp.int32(1))[1],
                       jnp.int32(0))

        for t in range(tcb):
            pltpu.sync_copy(rows_v.at[pl.ds(t * (nchunk // tcb),
                                            nchunk // tcb)],
                            out_hbm.at[jnp.int32(t), wid])

    return grid_hash_gather


def kernel(positions, table):
    n = positions.shape[0]
    hash_size, dim = table.shape
    tflat = table.reshape(hash_size // 128, 128, dim // 8, 8)
    tflat = tflat.transpose(2, 0, 3, 1).reshape(-1)
    px = positions[:, 0]
    py = positions[:, 1]
    out = _build(n, hash_size, dim)(tflat, px, py)
    tcb = dim // 8
    out = out.reshape(tcb, n // 512, 4, 8, 128)
    out = out.transpose(1, 2, 4, 0, 3).reshape(n, dim)
    return out


# speedup vs baseline: 10.2000x; 1.0307x over previous
import functools

import jax
import jax.numpy as jnp
from jax import lax
from jax.experimental import pallas as pl
from jax.experimental.pallas import tpu as pltpu
from jax.experimental.pallas import tpu_sc as plsc

_C1 = 2246822507
_C2 = 3266489909


def _u32(v):
    return jnp.uint32(v)


def _mulhi_u32(a, b_const):
    a0 = a & _u32(0xFFFF)
    a1 = a >> _u32(16)
    b0 = _u32(b_const & 0xFFFF)
    b1 = _u32(b_const >> 16)
    t = a0 * b0
    t1 = a1 * b0 + (t >> _u32(16))
    t2 = a0 * b1 + (t1 & _u32(0xFFFF))
    return a1 * b1 + (t1 >> _u32(16)) + (t2 >> _u32(16))


def _floor_i32(x):
    t = x.astype(jnp.int32)
    tf = t.astype(jnp.float32)
    return t - jnp.where(tf > x, jnp.int32(1), jnp.int32(0))


def _bitmix_idx(x, y, mask):
    ix = _floor_i32(x)
    iy = _floor_i32(y)
    ixu = ix.astype(jnp.uint32)
    iyu = iy.astype(jnp.uint32)
    s = (ix >> 31).astype(jnp.uint32)
    d = ixu ^ ((ixu >> _u32(16)) | (s << _u32(16)))
    e_lo = d * _u32(_C1)
    e_hi = _mulhi_u32(d, _C1)
    f_lo = e_lo ^ ((e_lo >> _u32(13)) | (e_hi << _u32(19)))
    f_hi = e_hi ^ (e_hi >> _u32(13))
    t_lo = iyu * _u32(_C2)
    t_hi = _mulhi_u32(iyu, _C2) + jnp.where(
        iy < 0, _u32((1 << 32) - _C2), _u32(0))
    g_lo = f_lo + t_lo
    carry = jnp.where(g_lo < t_lo, _u32(1), _u32(0))
    g_hi = f_hi + t_hi + carry
    out = (g_lo ^ ((g_lo >> _u32(16)) | (g_hi << _u32(16)))) & _u32(mask)
    return out.astype(jnp.int32)


@functools.cache
def _build(n, hash_size, dim):
    assert hash_size & (hash_size - 1) == 0
    info = plsc.get_sparse_core_info()
    nc, ns, lanes = info.num_cores, info.num_subcores, info.num_lanes
    nw = nc * ns
    b_per_w = n // nw
    assert b_per_w * nw == n and lanes == 16 and dim == 64 and b_per_w == 512
    nchunk = b_per_w * dim // 128
    trw = b_per_w // 128
    tcb = dim // 8

    mesh = plsc.VectorSubcoreMesh(core_axis_name="c", subcore_axis_name="s")

    @functools.partial(
        pl.kernel,
        out_type=jax.ShapeDtypeStruct((tcb, nw, trw * 8, 128), jnp.float32),
        mesh=mesh,
        compiler_params=pltpu.CompilerParams(
            use_tc_tiling_on_sc=False, needs_layout_passes=False),
        scratch_types=[
            pltpu.VMEM((b_per_w,), jnp.float32),
            pltpu.VMEM((b_per_w,), jnp.float32),
            pltpu.VMEM((b_per_w,), jnp.int32),
            pltpu.VMEM((nchunk, 128), jnp.int32),
            pltpu.VMEM((nchunk, 128), jnp.float32),
            pltpu.VMEM((512,), jnp.float32),
            pltpu.SemaphoreType.DMA,
        ],
    )
    def grid_hash_gather(tflat_hbm, px_hbm, py_hbm, out_hbm,
                         px_v, py_v, b_v, idx_v, rows_v, drain_v, sem0):
        wid = lax.axis_index("s") * nc + lax.axis_index("c")
        base = wid * b_per_w
        pltpu.sync_copy(px_hbm.at[pl.ds(base, b_per_w)], px_v)
        pltpu.sync_copy(py_hbm.at[pl.ds(base, b_per_w)], py_v)

        for i in range(b_per_w // lanes):
            x = px_v[pl.ds(i * lanes, lanes)]
            y = py_v[pl.ds(i * lanes, lanes)]
            r = _bitmix_idx(x, y, hash_size - 1)
            b_v[pl.ds(i * lanes, lanes)] = (
                ((r >> jnp.int32(7)) << jnp.int32(10))
                | (r & jnp.int32(127)))

        def compute_idx(k):
            ci = k & jnp.int32(7)
            tr_loc = (k >> jnp.int32(3)) & jnp.int32(trw - 1)
            tc = k >> jnp.int32(5)
            cc = (tc << jnp.int32(23)) | (ci << jnp.int32(7))
            ccv = jnp.full((lanes,), jnp.int32(0), jnp.int32) + cc
            pbase = tr_loc * jnp.int32(128)
            for u in range(128 // lanes):
                bv = b_v[pl.ds(pbase + jnp.int32(u * lanes), lanes)]
                idx_v[k, pl.ds(u * lanes, lanes)] = bv + ccv

        def fire(ch):
            pltpu.make_async_copy(
                tflat_hbm.at[idx_v.at[ch]], rows_v.at[ch], sem0).start()

        def wait_four():
            pltpu.make_async_copy(
                px_hbm.at[pl.ds(jnp.int32(0), 512)], drain_v, sem0).wait()

        depth = nchunk

        def prime_body(ch):
            compute_idx(ch)
            fire(ch)
            return ch + jnp.int32(1)

        lax.while_loop(lambda ch: ch < jnp.int32(depth), prime_body,
                       jnp.int32(0))

        def stream_body(ch):
            for u in range(4):
                c = ch + jnp.int32(u)
                compute_idx(c)
                fire(c)
            wait_four()
            return ch + jnp.int32(4)

        lax.while_loop(lambda ch: ch < jnp.int32(nchunk), stream_body,
                       jnp.int32(depth))

        lax.while_loop(lambda ch: ch < jnp.int32(depth // 4),
                       lambda ch: (wait_four(), ch + jnp.int32(1))[1],
                       jnp.int32(0))

        for t in range(tcb):
            pltpu.sync_copy(rows_v.at[pl.ds(t * (nchunk // tcb),
                                            nchunk // tcb)],
                            out_hbm.at[jnp.int32(t), wid])

    return grid_hash_gather


def kernel(positions, table):
    n = positions.shape[0]
    hash_size, dim = table.shape
    tflat = table.reshape(hash_size // 128, 128, dim // 8, 8)
    tflat = tflat.transpose(2, 0, 3, 1).reshape(-1)
    px = positions[:, 0]
    py = positions[:, 1]
    out = _build(n, hash_size, dim)(tflat, px, py)
    tcb = dim // 8
    out = out.reshape(tcb, n // 512, 4, 8, 128)
    out = out.transpose(1, 2, 4, 0, 3).reshape(n, dim)
    return out
